# Initial kernel scaffold; baseline (speedup 1.0000x reference)
import jax
import jax.numpy as jnp
from jax.experimental import pallas as pl


def kernel(inputs):
    raise NotImplementedError("write your pallas kernel here")


# speedup vs baseline: 1.0537x; 1.0537x over previous
import functools

import jax
import jax.numpy as jnp
from jax import lax
from jax.experimental import pallas as pl
from jax.experimental.pallas import tpu as pltpu
from jax.experimental.pallas import tpu_sc as plsc

N_ROWS = 16384
N_COLS = 256
K_OUT = 128
M_OUT = N_ROWS * K_OUT
NUM_WORKERS = 32
PER_WORKER = M_OUT // NUM_WORKERS
BLK = 16384
NBLK = PER_WORKER // BLK
LANES = 16
UNROLL = 8


def _build_sc_kernel():
    mesh = plsc.VectorSubcoreMesh(core_axis_name="c", subcore_axis_name="s")

    @functools.partial(
        pl.kernel,
        mesh=mesh,
        out_type=jax.ShapeDtypeStruct((M_OUT,), jnp.float32),
        compiler_params=pltpu.CompilerParams(needs_layout_passes=False),
        scratch_types=[
            pltpu.VMEM((2 * BLK,), jnp.float32),
            pltpu.VMEM((BLK,), jnp.float32),
        ],
    )
    def k(in_hbm, out_hbm, in_v, out_v):
        wid = lax.axis_index("s") * 2 + lax.axis_index("c")
        lane2 = 2 * lax.iota(jnp.int32, LANES)

        for b in range(NBLK):
            obase = wid * PER_WORKER + b * BLK
            pltpu.sync_copy(in_hbm.at[pl.ds(2 * obase, 2 * BLK)], in_v)

            def body(g, _):
                mbase = g * (LANES * UNROLL)
                for j in range(UNROLL):
                    idx = lane2 + (2 * mbase + 2 * LANES * j)
                    v = plsc.load_gather(in_v, [idx])
                    out_v[pl.ds(mbase + LANES * j, LANES)] = v
                return 0

            lax.fori_loop(0, BLK // (LANES * UNROLL), body, 0)
            pltpu.sync_copy(out_v, out_hbm.at[pl.ds(obase, BLK)])

    return k


_SC_KERNEL = _build_sc_kernel()


def kernel(inputs):
    flat = inputs.reshape(-1)
    out = _SC_KERNEL(flat)
    return out.reshape(N_ROWS, K_OUT)


# speedup vs baseline: 1.4255x; 1.3528x over previous
import functools

import jax
import jax.numpy as jnp
from jax import lax
from jax.experimental import pallas as pl
from jax.experimental.pallas import tpu as pltpu
from jax.experimental.pallas import tpu_sc as plsc

N_ROWS = 16384
N_COLS = 256
K_OUT = 128
NUM_WORKERS = 32
ROWS_PER_WORKER = N_ROWS // NUM_WORKERS
BLK_ROWS = 128
NBLK = ROWS_PER_WORKER // BLK_ROWS
LANES = 16


def _build_sc_kernel():
    mesh = plsc.VectorSubcoreMesh(core_axis_name="c", subcore_axis_name="s")

    @functools.partial(
        pl.kernel,
        mesh=mesh,
        out_type=jax.ShapeDtypeStruct((N_ROWS, K_OUT), jnp.float32),
        compiler_params=pltpu.CompilerParams(
            needs_layout_passes=False,
            use_tc_tiling_on_sc=True,
        ),
        scratch_types=[
            pltpu.VMEM((BLK_ROWS, N_COLS), jnp.float32),
            pltpu.VMEM((BLK_ROWS, K_OUT), jnp.float32),
        ],
    )
    def k(in_hbm, out_hbm, in_v, out_v):
        wid = lax.axis_index("s") * 2 + lax.axis_index("c")
        lane2 = 2 * lax.iota(jnp.int32, LANES)

        for b in range(NBLK):
            row0 = wid * ROWS_PER_WORKER + b * BLK_ROWS
            pltpu.sync_copy(in_hbm.at[pl.ds(row0, BLK_ROWS), :], in_v)

            def body(r, _):
                rows = jnp.full((LANES,), r, jnp.int32)
                for t in range(K_OUT // LANES):
                    cols = lane2 + (2 * LANES * t)
                    v = plsc.load_gather(in_v, [rows, cols])
                    out_v[r, pl.ds(LANES * t, LANES)] = v
                return 0

            lax.fori_loop(0, BLK_ROWS, body, 0)
            pltpu.sync_copy(out_v, out_hbm.at[pl.ds(row0, BLK_ROWS), :])

    return k


_SC_KERNEL = _build_sc_kernel()


def kernel(inputs):
    return _SC_KERNEL(inputs)


# speedup vs baseline: 1.6801x; 1.1787x over previous
import functools

import jax
import jax.numpy as jnp
from jax import lax
from jax.experimental import pallas as pl
from jax.experimental.pallas import tpu as pltpu
from jax.experimental.pallas import tpu_sc as plsc

N_ROWS = 16384
N_COLS = 256
K_OUT = 128
NUM_WORKERS = 32
ROWS_PER_WORKER = N_ROWS // NUM_WORKERS
BLK_ROWS = 128
NBLK = ROWS_PER_WORKER // BLK_ROWS
LANES = 16


def _build_sc_kernel():
    mesh = plsc.VectorSubcoreMesh(core_axis_name="c", subcore_axis_name="s")

    @functools.partial(
        pl.kernel,
        mesh=mesh,
        out_type=jax.ShapeDtypeStruct((N_ROWS, K_OUT), jnp.float32),
        compiler_params=pltpu.CompilerParams(
            needs_layout_passes=False,
            use_tc_tiling_on_sc=True,
        ),
        scratch_types=[
            pltpu.VMEM((2, BLK_ROWS, N_COLS), jnp.float32),
            pltpu.VMEM((2, BLK_ROWS, K_OUT), jnp.float32),
            pltpu.SemaphoreType.DMA((2,)),
            pltpu.SemaphoreType.DMA((2,)),
        ],
    )
    def k(in_hbm, out_hbm, in_v, out_v, in_sem, out_sem):
        wid = lax.axis_index("s") * 2 + lax.axis_index("c")
        lane2 = 2 * lax.iota(jnp.int32, LANES)
        cols = [lane2 + (2 * LANES * t) for t in range(K_OUT // LANES)]

        def row0(b):
            return wid * ROWS_PER_WORKER + b * BLK_ROWS

        def start_in(b):
            return pltpu.async_copy(
                in_hbm.at[pl.ds(row0(b), BLK_ROWS), :],
                in_v.at[b % 2],
                in_sem.at[b % 2],
            )

        def start_out(b):
            return pltpu.async_copy(
                out_v.at[b % 2],
                out_hbm.at[pl.ds(row0(b), BLK_ROWS), :],
                out_sem.at[b % 2],
            )

        in_copies = {0: start_in(0)}
        out_copies = {}
        for b in range(NBLK):
            if b + 1 < NBLK:
                in_copies[b + 1] = start_in(b + 1)
            in_copies.pop(b).wait()
            if b >= 2:
                out_copies.pop(b - 2).wait()

            src = in_v.at[b % 2]
            dst = out_v.at[b % 2]

            def body(r, _):
                rows = jnp.full((LANES,), r, jnp.int32)
                for t in range(K_OUT // LANES):
                    v = plsc.load_gather(src, [rows, cols[t]])
                    dst[r, pl.ds(LANES * t, LANES)] = v
                return 0

            lax.fori_loop(0, BLK_ROWS, body, 0)
            out_copies[b] = start_out(b)
        for b in sorted(out_copies):
            out_copies.pop(b).wait()

    return k


_SC_KERNEL = _build_sc_kernel()


def kernel(inputs):
    return _SC_KERNEL(inputs)


# speedup vs baseline: 2.1784x; 1.2966x over previous
import functools

import jax
import jax.numpy as jnp
from jax import lax
from jax.experimental import pallas as pl
from jax.experimental.pallas import tpu as pltpu
from jax.experimental.pallas import tpu_sc as plsc

N_ROWS = 16384
N_COLS = 256
K_OUT = 128
NUM_WORKERS = 32
ROWS_PER_WORKER = N_ROWS // NUM_WORKERS
BLK_ROWS = 128
NBLK = ROWS_PER_WORKER // BLK_ROWS
LANES = 16


def _build_sc_kernel():
    mesh = plsc.VectorSubcoreMesh(core_axis_name="c", subcore_axis_name="s")

    @functools.partial(
        pl.kernel,
        mesh=mesh,
        out_type=jax.ShapeDtypeStruct((N_ROWS, K_OUT), jnp.float32),
        compiler_params=pltpu.CompilerParams(
            needs_layout_passes=False,
            use_tc_tiling_on_sc=True,
        ),
        scratch_types=[
            pltpu.VMEM((2, BLK_ROWS, N_COLS), jnp.float32),
            pltpu.VMEM((2, BLK_ROWS, K_OUT), jnp.float32),
            pltpu.SemaphoreType.DMA((2,)),
            pltpu.SemaphoreType.DMA((2,)),
        ],
    )
    def k(in_hbm, out_hbm, in_v, out_v, in_sem, out_sem):
        wid = lax.axis_index("s") * 2 + lax.axis_index("c")
        lane2 = 2 * lax.iota(jnp.int32, LANES)
        cols = [lane2 + (2 * LANES * t) for t in range(K_OUT // LANES)]

        def row0(b):
            return wid * ROWS_PER_WORKER + b * BLK_ROWS

        def start_in(b):
            return pltpu.async_copy(
                in_hbm.at[pl.ds(row0(b), BLK_ROWS), :],
                in_v.at[b % 2],
                in_sem.at[b % 2],
            )

        def start_out(b):
            return pltpu.async_copy(
                out_v.at[b % 2],
                out_hbm.at[pl.ds(row0(b), BLK_ROWS), :],
                out_sem.at[b % 2],
            )

        in_copies = {0: start_in(0)}
        out_copies = {}
        for b in range(NBLK):
            if b + 1 < NBLK:
                in_copies[b + 1] = start_in(b + 1)
            in_copies.pop(b).wait()
            if b >= 2:
                out_copies.pop(b - 2).wait()

            src = in_v.at[b % 2]
            dst = out_v.at[b % 2]

            @plsc.parallel_loop(0, BLK_ROWS, unroll=4)
            def body(r):
                rows = jnp.full((LANES,), r, jnp.int32)
                for t in range(K_OUT // LANES):
                    v = plsc.load_gather(src, [rows, cols[t]])
                    dst[r, pl.ds(LANES * t, LANES)] = v
            out_copies[b] = start_out(b)
        for b in sorted(out_copies):
            out_copies.pop(b).wait()

    return k


_SC_KERNEL = _build_sc_kernel()


def kernel(inputs):
    return _SC_KERNEL(inputs)
